# baseline (speedup 1.0000x reference)
import functools

import jax
import jax.numpy as jnp
from jax import lax
from jax.experimental import pallas as pl
from jax.experimental.pallas import tpu as pltpu
from jax.experimental.pallas import tpu_sc as plsc

FEAT_DIM = 256
NUM_HEADS = 8
HEAD_DIM = FEAT_DIM // NUM_HEADS
NUM_POS = 128
NUM_NEG = 128
NUM_SAMPLES = NUM_POS + NUM_NEG
HW = 4096
BATCH = 8
N_SLOTS = 800
_ROWS = HW * BATCH
_BLK = 4096
_PACKED = FEAT_DIM // 2

_SC_CORES = 2
_SC_SUBCORES = 16
_NW = _SC_CORES * _SC_SUBCORES
_SLOTS_PER_W = N_SLOTS // _NW
_SCALING = float(HEAD_DIM) ** -0.5


def _pack_tab(xf):
    u1 = jax.lax.bitcast_convert_type(xf[:, :_PACKED], jnp.int32)
    u2 = jax.lax.bitcast_convert_type(xf[:, _PACKED:], jnp.int32)
    lo = jax.lax.shift_right_logical(u1 + 0x8000, 16)
    hi = (u2 + 0x8000) & jnp.int32(-65536)
    return lo | hi


def _tc1_body(ff_ref, pf_ref, wkT_ref, bk_ref, wvT_ref, bv_ref,
              slots_ref, wqT_ref, bq_ref, ktab_ref, vtab_ref, q_ref):
    x = ff_ref[...]
    kf = jnp.dot((x + pf_ref[...]).astype(jnp.bfloat16),
                 wkT_ref[...].astype(jnp.bfloat16),
                 preferred_element_type=jnp.float32) + bk_ref[...]
    vf = jnp.dot(x.astype(jnp.bfloat16), wvT_ref[...].astype(jnp.bfloat16),
                 preferred_element_type=jnp.float32) + bv_ref[...]
    ktab_ref[...] = _pack_tab(kf)
    vtab_ref[...] = _pack_tab(vf)

    @pl.when(pl.program_id(0) == 0)
    def _():
        q_ref[...] = (jnp.dot(slots_ref[...], wqT_ref[...],
                              preferred_element_type=jnp.float32)
                      + bq_ref[...]) * _SCALING


def _tc1(ff, pf, wkT, bk, wvT, bv, slots2d, wqT, bq):
    full = lambda shape: pl.BlockSpec(shape, lambda i: (0, 0))
    return pl.pallas_call(
        _tc1_body,
        grid=(_ROWS // _BLK,),
        in_specs=[
            pl.BlockSpec((_BLK, FEAT_DIM), lambda i: (i, 0)),
            pl.BlockSpec((_BLK, FEAT_DIM), lambda i: (i, 0)),
            full((FEAT_DIM, FEAT_DIM)),
            full((1, FEAT_DIM)),
            full((FEAT_DIM, FEAT_DIM)),
            full((1, FEAT_DIM)),
            full((N_SLOTS, FEAT_DIM)),
            full((FEAT_DIM, FEAT_DIM)),
            full((1, FEAT_DIM)),
        ],
        out_specs=[
            pl.BlockSpec((_BLK, _PACKED), lambda i: (i, 0)),
            pl.BlockSpec((_BLK, _PACKED), lambda i: (i, 0)),
            full((N_SLOTS, FEAT_DIM)),
        ],
        out_shape=[jax.ShapeDtypeStruct((_ROWS, _PACKED), jnp.int32),
                   jax.ShapeDtypeStruct((_ROWS, _PACKED), jnp.int32),
                   jax.ShapeDtypeStruct((N_SLOTS, FEAT_DIM), jnp.float32)],
    )(ff, pf, wkT, bk, wvT, bv, slots2d, wqT, bq)


def _sc_attention(ktab, vtab, q, fidx):
    mesh = plsc.VectorSubcoreMesh(core_axis_name="c", subcore_axis_name="s",
                                  num_cores=_SC_CORES,
                                  num_subcores=_SC_SUBCORES)

    @functools.partial(
        pl.kernel,
        out_type=[jax.ShapeDtypeStruct((N_SLOTS, FEAT_DIM), jnp.float32),
                  jax.ShapeDtypeStruct((N_SLOTS, NUM_SAMPLES), jnp.float32)],
        mesh=mesh,
        compiler_params=pltpu.CompilerParams(use_tc_tiling_on_sc=False),
        scratch_types=[
            pltpu.VMEM((_SLOTS_PER_W, 2, NUM_POS), jnp.int32),
            pltpu.VMEM((_SLOTS_PER_W, FEAT_DIM), jnp.float32),
            pltpu.VMEM((2, NUM_POS, _PACKED), jnp.int32),
            pltpu.VMEM((2, NUM_NEG, _PACKED), jnp.int32),
            pltpu.VMEM((2, 64, _PACKED), jnp.int32),
            pltpu.VMEM((2, 64, _PACKED), jnp.int32),
            pltpu.VMEM((16, NUM_HEADS, 16), jnp.float32),
            pltpu.VMEM((NUM_SAMPLES,), jnp.float32),
            pltpu.VMEM((FEAT_DIM,), jnp.float32),
            pltpu.SemaphoreType.DMA,
            pltpu.SemaphoreType.DMA,
            pltpu.SemaphoreType.DMA,
            pltpu.SemaphoreType.DMA,
            pltpu.SemaphoreType.DMA,
            pltpu.SemaphoreType.DMA,
        ],
    )
    def sc_kernel(ktab_hbm, vtab_hbm, q_hbm, fidx_hbm, attn_hbm, naff_hbm,
                  fidx_all_v, q_all_v, kbufA_v, kbufB_v, vbufA_v, vbufB_v,
                  aff_v, naff_v, out_v,
                  sem_k0, sem_k1, sem_v0, sem_v1, sem_no, sem_ao):
        wid = lax.axis_index("s") * _SC_CORES + lax.axis_index("c")
        base_slot = wid * _SLOTS_PER_W

        io16 = jnp.arange(16, dtype=jnp.int32)
        himask = jnp.full((16,), -65536, jnp.int32)

        def _wlo(iv):
            return jax.lax.bitcast_convert_type(iv << 16, jnp.float32)

        def _whi(iv):
            return jax.lax.bitcast_convert_type(iv & himask, jnp.float32)

        def _shuf(v, k):
            return jnp.take_along_axis(v, io16 ^ k, axis=0)

        def _merge(a, b, k):
            m = (io16 & k) == 0
            return jnp.where(m, a, b) + _shuf(jnp.where(m, b, a), k)

        def _butterfly_push(stack, v):
            lvl = 1
            while stack and stack[-1][0] == lvl:
                _, a = stack.pop()
                v = _merge(a, v, lvl)
                lvl <<= 1
            stack.append((lvl, v))
            return stack

        def _butterfly_sum(vs):
            stack = []
            for v in vs:
                stack = _butterfly_push(stack, v)
            (_, out), = stack
            return out

        def _allreduce_sum(v):
            for k in (1, 2, 4, 8):
                v = v + _shuf(v, k)
            return v

        def issue_k0(i, b):
            return pltpu.make_async_copy(
                ktab_hbm.at[fidx_all_v.at[i, 0]], kbufA_v.at[b], sem_k0)

        def issue_k1(i, b):
            return pltpu.make_async_copy(
                ktab_hbm.at[fidx_all_v.at[i, 1]], kbufB_v.at[b], sem_k1)

        def issue_v0(i, b):
            return pltpu.make_async_copy(
                vtab_hbm.at[fidx_all_v.at[i, 0, pl.ds(0, 64)]],
                vbufA_v.at[b], sem_v0)

        def issue_v1(i, b):
            return pltpu.make_async_copy(
                vtab_hbm.at[fidx_all_v.at[i, 0, pl.ds(64, 64)]],
                vbufB_v.at[b], sem_v1)

        pltpu.sync_copy(fidx_hbm.at[pl.ds(base_slot, _SLOTS_PER_W)],
                        fidx_all_v)
        pltpu.sync_copy(q_hbm.at[pl.ds(base_slot, _SLOTS_PER_W)], q_all_v)
        issue_k0(0, 0).start()
        issue_k1(0, 0).start()
        issue_v0(0, 0).start()
        issue_v1(0, 0).start()

        def slot_body(i, carry):
            n = base_slot + i
            last = _SLOTS_PER_W - 1
            b0 = i & 1
            b1 = 1 - b0

            @pl.when(i > 0)
            def _():
                pltpu.make_async_copy(naff_v, naff_hbm.at[n - 1],
                                      sem_no).wait()
                pltpu.make_async_copy(out_v, attn_hbm.at[n - 1],
                                      sem_ao).wait()

            @pl.when(i < last)
            def _():
                issue_k0(i + 1, b1).start()
                issue_k1(i + 1, b1).start()
                issue_v0(i + 1, b1).start()
                issue_v1(i + 1, b1).start()

            qv = [q_all_v[i, pl.ds(c * 16, 16)] for c in range(16)]

            def aff_group_for(buf, goff):
                def aff_group(g, _):
                    base = g * 16
                    for hp in range(4):
                        lo_stack, hi_stack = [], []
                        for lane in range(16):
                            s = base + lane
                            ivA = buf[b0, s, pl.ds(hp * 32, 16)]
                            ivB = buf[b0, s, pl.ds(hp * 32 + 16, 16)]
                            plo = _wlo(ivA) * qv[2 * hp] + \
                                _wlo(ivB) * qv[2 * hp + 1]
                            phi = _whi(ivA) * qv[2 * hp + 8] + \
                                _whi(ivB) * qv[2 * hp + 9]
                            lo_stack = _butterfly_push(lo_stack, plo)
                            hi_stack = _butterfly_push(hi_stack, phi)
                        aff_v[g + goff, hp] = lo_stack[0][1]
                        aff_v[g + goff, hp + 4] = hi_stack[0][1]
                    return 0
                return aff_group

            issue_k0(i, b0).wait()
            lax.fori_loop(0, 8, aff_group_for(kbufA_v, 0), 0)
            issue_k1(i, b0).wait()
            lax.fori_loop(0, 8, aff_group_for(kbufB_v, 8), 0)

            nacc = [jnp.zeros((16,), jnp.float32) for _ in range(16)]
            for h in range(NUM_HEADS):
                evecs = [jnp.exp(aff_v[g, h]) for g in range(16)]
                ssum = evecs[0]
                for g in range(1, 16):
                    ssum = ssum + evecs[g]
                inv = 1.0 / _allreduce_sum(ssum)
                for g in range(16):
                    w = evecs[g] * inv
                    aff_v[g, h] = w
                    nacc[g] = nacc[g] + w
            for g in range(16):
                naff_v[pl.ds(g * 16, 16)] = nacc[g] * (1.0 / NUM_HEADS)
            pltpu.async_copy(naff_v, naff_hbm.at[n], sem_no)

            def val_group_for(buf, goff):
                def val_group(g, acc):
                    accl = list(acc)
                    for hp in range(4):
                        wv_lo = aff_v[g + goff, hp]
                        wv_hi = aff_v[g + goff, hp + 4]
                        for lane in range(16):
                            s = g * 16 + lane
                            ivA = buf[b0, s, pl.ds(hp * 32, 16)]
                            ivB = buf[b0, s, pl.ds(hp * 32 + 16, 16)]
                            wl = wv_lo[lane]
                            wh = wv_hi[lane]
                            accl[2 * hp] = accl[2 * hp] + _wlo(ivA) * wl
                            accl[2 * hp + 1] = \
                                accl[2 * hp + 1] + _wlo(ivB) * wl
                            accl[2 * hp + 8] = \
                                accl[2 * hp + 8] + _whi(ivA) * wh
                            accl[2 * hp + 9] = \
                                accl[2 * hp + 9] + _whi(ivB) * wh
                    return tuple(accl)
                return val_group

            acc0 = tuple(jnp.zeros((16,), jnp.float32) for _ in range(16))
            issue_v0(i, b0).wait()
            acc = lax.fori_loop(0, 4, val_group_for(vbufA_v, 0), acc0)
            issue_v1(i, b0).wait()
            acc = lax.fori_loop(0, 4, val_group_for(vbufB_v, 4), acc)

            for c in range(16):
                out_v[pl.ds(c * 16, 16)] = acc[c]
            pltpu.async_copy(out_v, attn_hbm.at[n], sem_ao)
            return carry

        lax.fori_loop(0, _SLOTS_PER_W, slot_body, 0)
        pltpu.make_async_copy(
            naff_v, naff_hbm.at[base_slot + _SLOTS_PER_W - 1], sem_no).wait()
        pltpu.make_async_copy(
            out_v, attn_hbm.at[base_slot + _SLOTS_PER_W - 1], sem_ao).wait()

    return sc_kernel(ktab, vtab, q, fidx)


def _tc3_body(slots_ref, attn_ref, woT_ref, bo_ref, g_ref, b_ref, naff_ref,
              new_ref, nafft_ref):
    y = slots_ref[...] + jnp.dot(attn_ref[...], woT_ref[...],
                                 preferred_element_type=jnp.float32) + bo_ref[...]
    mu = jnp.mean(y, axis=-1, keepdims=True)
    var = jnp.mean((y - mu) ** 2, axis=-1, keepdims=True)
    new_ref[...] = (y - mu) / jnp.sqrt(var + 1e-5) * g_ref[...] + b_ref[...]
    nafft_ref[...] = naff_ref[...].T


def _tc3(slots2d, attn2d, woT, bo, gamma, beta, naff):
    return pl.pallas_call(
        _tc3_body,
        out_shape=[jax.ShapeDtypeStruct((N_SLOTS, FEAT_DIM), jnp.float32),
                   jax.ShapeDtypeStruct((NUM_SAMPLES, N_SLOTS), jnp.float32)],
    )(slots2d, attn2d, woT, bo, gamma, beta, naff)


def kernel(slots, features, pos_encodings, feat_idx, batch_idx, in_proj_weight,
           in_proj_bias, out_proj_w, out_proj_b, ln_gamma, ln_beta):
    D = FEAT_DIM
    ff = features.reshape(_ROWS, D)
    pf = pos_encodings.reshape(_ROWS, D)
    wqT = in_proj_weight[:D].T
    wkT = in_proj_weight[D:2 * D].T
    wvT = in_proj_weight[2 * D:].T
    bq = in_proj_bias[:D].reshape(1, D)
    bk = in_proj_bias[D:2 * D].reshape(1, D)
    bv = in_proj_bias[2 * D:].reshape(1, D)
    slots2d = slots.reshape(N_SLOTS, D)
    fidx = (feat_idx.astype(jnp.int32) * BATCH
            + batch_idx[None, :].astype(jnp.int32)).T.reshape(
                N_SLOTS, 2, NUM_POS)

    ktab, vtab, q = _tc1(ff, pf, wkT, bk, wvT, bv, slots2d, wqT, bq)
    attn, naff = _sc_attention(ktab, vtab, q, fidx)
    new2d, norm_aff = _tc3(slots2d, attn, out_proj_w.T,
                           out_proj_b.reshape(1, D), ln_gamma.reshape(1, D),
                           ln_beta.reshape(1, D), naff)
    return new2d.reshape(1, N_SLOTS, D), norm_aff


# speedup vs baseline: 1.0623x; 1.0623x over previous
import functools

import jax
import jax.numpy as jnp
from jax import lax
from jax.experimental import pallas as pl
from jax.experimental.pallas import tpu as pltpu
from jax.experimental.pallas import tpu_sc as plsc

FEAT_DIM = 256
NUM_HEADS = 8
HEAD_DIM = FEAT_DIM // NUM_HEADS
NUM_POS = 128
NUM_NEG = 128
NUM_SAMPLES = NUM_POS + NUM_NEG
HW = 4096
BATCH = 8
N_SLOTS = 800
_ROWS = HW * BATCH
_BLK = 4096
_PACKED = FEAT_DIM // 2

_SC_CORES = 2
_SC_SUBCORES = 16
_NW = _SC_CORES * _SC_SUBCORES
_SLOTS_PER_W = N_SLOTS // _NW
_SCALING = float(HEAD_DIM) ** -0.5


def _pack_tab(xf):
    u1 = jax.lax.bitcast_convert_type(xf[:, :_PACKED], jnp.int32)
    u2 = jax.lax.bitcast_convert_type(xf[:, _PACKED:], jnp.int32)
    lo = jax.lax.shift_right_logical(u1 + 0x8000, 16)
    hi = (u2 + 0x8000) & jnp.int32(-65536)
    return lo | hi


def _tc1_body(ff_ref, pf_ref, wkT_ref, bk_ref, wvT_ref, bv_ref,
              slots_ref, wqT_ref, bq_ref, ktab_ref, vtab_ref, q_ref):
    x = ff_ref[...]
    kf = jnp.dot((x + pf_ref[...]).astype(jnp.bfloat16),
                 wkT_ref[...].astype(jnp.bfloat16),
                 preferred_element_type=jnp.float32) + bk_ref[...]
    vf = jnp.dot(x.astype(jnp.bfloat16), wvT_ref[...].astype(jnp.bfloat16),
                 preferred_element_type=jnp.float32) + bv_ref[...]
    ktab_ref[...] = _pack_tab(kf)
    vtab_ref[...] = _pack_tab(vf)

    @pl.when(pl.program_id(0) == 0)
    def _():
        q_ref[...] = (jnp.dot(slots_ref[...], wqT_ref[...],
                              preferred_element_type=jnp.float32)
                      + bq_ref[...]) * _SCALING


def _tc1(ff, pf, wkT, bk, wvT, bv, slots2d, wqT, bq):
    full = lambda shape: pl.BlockSpec(shape, lambda i: (0, 0))
    return pl.pallas_call(
        _tc1_body,
        grid=(_ROWS // _BLK,),
        in_specs=[
            pl.BlockSpec((_BLK, FEAT_DIM), lambda i: (i, 0)),
            pl.BlockSpec((_BLK, FEAT_DIM), lambda i: (i, 0)),
            full((FEAT_DIM, FEAT_DIM)),
            full((1, FEAT_DIM)),
            full((FEAT_DIM, FEAT_DIM)),
            full((1, FEAT_DIM)),
            full((N_SLOTS, FEAT_DIM)),
            full((FEAT_DIM, FEAT_DIM)),
            full((1, FEAT_DIM)),
        ],
        out_specs=[
            pl.BlockSpec((_BLK, _PACKED), lambda i: (i, 0)),
            pl.BlockSpec((_BLK, _PACKED), lambda i: (i, 0)),
            full((N_SLOTS, FEAT_DIM)),
        ],
        out_shape=[jax.ShapeDtypeStruct((_ROWS, _PACKED), jnp.int32),
                   jax.ShapeDtypeStruct((_ROWS, _PACKED), jnp.int32),
                   jax.ShapeDtypeStruct((N_SLOTS, FEAT_DIM), jnp.float32)],
    )(ff, pf, wkT, bk, wvT, bv, slots2d, wqT, bq)


def _sc_attention(ktab, vtab, q, fidx):
    mesh = plsc.VectorSubcoreMesh(core_axis_name="c", subcore_axis_name="s",
                                  num_cores=_SC_CORES,
                                  num_subcores=_SC_SUBCORES)

    @functools.partial(
        pl.kernel,
        out_type=[jax.ShapeDtypeStruct((N_SLOTS, FEAT_DIM), jnp.float32),
                  jax.ShapeDtypeStruct((N_SLOTS, NUM_SAMPLES), jnp.float32)],
        mesh=mesh,
        compiler_params=pltpu.CompilerParams(use_tc_tiling_on_sc=False),
        scratch_types=[
            pltpu.VMEM((_SLOTS_PER_W, 2, NUM_POS), jnp.int32),
            pltpu.VMEM((_SLOTS_PER_W, FEAT_DIM), jnp.float32),
            pltpu.VMEM((2, NUM_POS, _PACKED), jnp.int32),
            pltpu.VMEM((2, NUM_NEG, _PACKED), jnp.int32),
            pltpu.VMEM((2, 64, _PACKED), jnp.int32),
            pltpu.VMEM((2, 64, _PACKED), jnp.int32),
            pltpu.VMEM((16, NUM_HEADS, 16), jnp.float32),
            pltpu.VMEM((NUM_SAMPLES,), jnp.float32),
            pltpu.VMEM((FEAT_DIM,), jnp.float32),
            pltpu.SemaphoreType.DMA,
            pltpu.SemaphoreType.DMA,
            pltpu.SemaphoreType.DMA,
            pltpu.SemaphoreType.DMA,
            pltpu.SemaphoreType.DMA,
            pltpu.SemaphoreType.DMA,
        ],
    )
    def sc_kernel(ktab_hbm, vtab_hbm, q_hbm, fidx_hbm, attn_hbm, naff_hbm,
                  fidx_all_v, q_all_v, kbufA_v, kbufB_v, vbufA_v, vbufB_v,
                  aff_v, naff_v, out_v,
                  sem_k0, sem_k1, sem_v0, sem_v1, sem_no, sem_ao):
        wid = lax.axis_index("s") * _SC_CORES + lax.axis_index("c")
        base_slot = wid * _SLOTS_PER_W

        io16 = jnp.arange(16, dtype=jnp.int32)
        himask = jnp.full((16,), -65536, jnp.int32)

        def _wlo(iv):
            return jax.lax.bitcast_convert_type(iv << 16, jnp.float32)

        def _whi(iv):
            return jax.lax.bitcast_convert_type(iv, jnp.float32)

        def _shuf(v, k):
            return jnp.take_along_axis(v, io16 ^ k, axis=0)

        def _merge(a, b, k):
            m = (io16 & k) == 0
            return jnp.where(m, a, b) + _shuf(jnp.where(m, b, a), k)

        def _butterfly_push(stack, v):
            lvl = 1
            while stack and stack[-1][0] == lvl:
                _, a = stack.pop()
                v = _merge(a, v, lvl)
                lvl <<= 1
            stack.append((lvl, v))
            return stack

        def _butterfly_sum(vs):
            stack = []
            for v in vs:
                stack = _butterfly_push(stack, v)
            (_, out), = stack
            return out

        def _allreduce_sum(v):
            for k in (1, 2, 4, 8):
                v = v + _shuf(v, k)
            return v

        def issue_k0(i, b):
            return pltpu.make_async_copy(
                ktab_hbm.at[fidx_all_v.at[i, 0]], kbufA_v.at[b], sem_k0)

        def issue_k1(i, b):
            return pltpu.make_async_copy(
                ktab_hbm.at[fidx_all_v.at[i, 1]], kbufB_v.at[b], sem_k1)

        def issue_v0(i, b):
            return pltpu.make_async_copy(
                vtab_hbm.at[fidx_all_v.at[i, 0, pl.ds(0, 64)]],
                vbufA_v.at[b], sem_v0)

        def issue_v1(i, b):
            return pltpu.make_async_copy(
                vtab_hbm.at[fidx_all_v.at[i, 0, pl.ds(64, 64)]],
                vbufB_v.at[b], sem_v1)

        pltpu.sync_copy(fidx_hbm.at[pl.ds(base_slot, _SLOTS_PER_W)],
                        fidx_all_v)
        pltpu.sync_copy(q_hbm.at[pl.ds(base_slot, _SLOTS_PER_W)], q_all_v)
        issue_k0(0, 0).start()
        issue_k1(0, 0).start()
        issue_v0(0, 0).start()
        issue_v1(0, 0).start()

        def slot_body(i, carry):
            n = base_slot + i
            last = _SLOTS_PER_W - 1
            b0 = i & 1
            b1 = 1 - b0

            @pl.when(i > 0)
            def _():
                pltpu.make_async_copy(naff_v, naff_hbm.at[n - 1],
                                      sem_no).wait()
                pltpu.make_async_copy(out_v, attn_hbm.at[n - 1],
                                      sem_ao).wait()

            @pl.when(i < last)
            def _():
                issue_k0(i + 1, b1).start()
                issue_k1(i + 1, b1).start()
                issue_v0(i + 1, b1).start()
                issue_v1(i + 1, b1).start()

            qv = [q_all_v[i, pl.ds(c * 16, 16)] for c in range(16)]

            def aff_group_for(buf, goff):
                def aff_group(g, _):
                    base = g * 16
                    for hp in range(4):
                        lo_stack, hi_stack = [], []
                        for lane in range(16):
                            s = base + lane
                            ivA = buf[b0, s, pl.ds(hp * 32, 16)]
                            ivB = buf[b0, s, pl.ds(hp * 32 + 16, 16)]
                            plo = _wlo(ivA) * qv[2 * hp] + \
                                _wlo(ivB) * qv[2 * hp + 1]
                            phi = _whi(ivA) * qv[2 * hp + 8] + \
                                _whi(ivB) * qv[2 * hp + 9]
                            lo_stack = _butterfly_push(lo_stack, plo)
                            hi_stack = _butterfly_push(hi_stack, phi)
                        aff_v[g + goff, hp] = lo_stack[0][1]
                        aff_v[g + goff, hp + 4] = hi_stack[0][1]
                    return 0
                return aff_group

            issue_k0(i, b0).wait()
            lax.fori_loop(0, 8, aff_group_for(kbufA_v, 0), 0)
            issue_k1(i, b0).wait()
            lax.fori_loop(0, 8, aff_group_for(kbufB_v, 8), 0)

            inv8 = []
            for h in range(NUM_HEADS):
                ssum = None
                for g in range(16):
                    e = jnp.exp(aff_v[g, h])
                    aff_v[g, h] = e
                    ssum = e if ssum is None else ssum + e
                inv8.append(1.0 / _allreduce_sum(ssum))
            invs = [inv8[h][0] for h in range(NUM_HEADS)]
            for g in range(16):
                nacc = None
                for h in range(NUM_HEADS):
                    w = aff_v[g, h] * inv8[h]
                    nacc = w if nacc is None else nacc + w
                naff_v[pl.ds(g * 16, 16)] = nacc * (1.0 / NUM_HEADS)
            pltpu.async_copy(naff_v, naff_hbm.at[n], sem_no)

            def val_group_for(buf, goff):
                def val_group(g, acc):
                    accl = list(acc)
                    for hp in range(4):
                        wv_lo = aff_v[g + goff, hp]
                        wv_hi = aff_v[g + goff, hp + 4]
                        for lane in range(16):
                            s = g * 16 + lane
                            ivA = buf[b0, s, pl.ds(hp * 32, 16)]
                            ivB = buf[b0, s, pl.ds(hp * 32 + 16, 16)]
                            wl = wv_lo[lane] * invs[hp]
                            wh = wv_hi[lane] * invs[hp + 4]
                            accl[2 * hp] = accl[2 * hp] + _wlo(ivA) * wl
                            accl[2 * hp + 1] = \
                                accl[2 * hp + 1] + _wlo(ivB) * wl
                            accl[2 * hp + 8] = \
                                accl[2 * hp + 8] + _whi(ivA) * wh
                            accl[2 * hp + 9] = \
                                accl[2 * hp + 9] + _whi(ivB) * wh
                    return tuple(accl)
                return val_group

            acc0 = tuple(jnp.zeros((16,), jnp.float32) for _ in range(16))
            issue_v0(i, b0).wait()
            acc = lax.fori_loop(0, 4, val_group_for(vbufA_v, 0), acc0)
            issue_v1(i, b0).wait()
            acc = lax.fori_loop(0, 4, val_group_for(vbufB_v, 4), acc)

            for c in range(16):
                out_v[pl.ds(c * 16, 16)] = acc[c]
            pltpu.async_copy(out_v, attn_hbm.at[n], sem_ao)
            return carry

        lax.fori_loop(0, _SLOTS_PER_W, slot_body, 0)
        pltpu.make_async_copy(
            naff_v, naff_hbm.at[base_slot + _SLOTS_PER_W - 1], sem_no).wait()
        pltpu.make_async_copy(
            out_v, attn_hbm.at[base_slot + _SLOTS_PER_W - 1], sem_ao).wait()

    return sc_kernel(ktab, vtab, q, fidx)


def _tc3_body(slots_ref, attn_ref, woT_ref, bo_ref, g_ref, b_ref, naff_ref,
              new_ref, nafft_ref):
    y = slots_ref[...] + jnp.dot(attn_ref[...], woT_ref[...],
                                 preferred_element_type=jnp.float32) + bo_ref[...]
    mu = jnp.mean(y, axis=-1, keepdims=True)
    var = jnp.mean((y - mu) ** 2, axis=-1, keepdims=True)
    new_ref[...] = (y - mu) / jnp.sqrt(var + 1e-5) * g_ref[...] + b_ref[...]
    nafft_ref[...] = naff_ref[...].T


def _tc3(slots2d, attn2d, woT, bo, gamma, beta, naff):
    return pl.pallas_call(
        _tc3_body,
        out_shape=[jax.ShapeDtypeStruct((N_SLOTS, FEAT_DIM), jnp.float32),
                   jax.ShapeDtypeStruct((NUM_SAMPLES, N_SLOTS), jnp.float32)],
    )(slots2d, attn2d, woT, bo, gamma, beta, naff)


def kernel(slots, features, pos_encodings, feat_idx, batch_idx, in_proj_weight,
           in_proj_bias, out_proj_w, out_proj_b, ln_gamma, ln_beta):
    D = FEAT_DIM
    ff = features.reshape(_ROWS, D)
    pf = pos_encodings.reshape(_ROWS, D)
    wqT = in_proj_weight[:D].T
    wkT = in_proj_weight[D:2 * D].T
    wvT = in_proj_weight[2 * D:].T
    bq = in_proj_bias[:D].reshape(1, D)
    bk = in_proj_bias[D:2 * D].reshape(1, D)
    bv = in_proj_bias[2 * D:].reshape(1, D)
    slots2d = slots.reshape(N_SLOTS, D)
    fidx = (feat_idx.astype(jnp.int32) * BATCH
            + batch_idx[None, :].astype(jnp.int32)).T.reshape(
                N_SLOTS, 2, NUM_POS)

    ktab, vtab, q = _tc1(ff, pf, wkT, bk, wvT, bv, slots2d, wqT, bq)
    attn, naff = _sc_attention(ktab, vtab, q, fidx)
    new2d, norm_aff = _tc3(slots2d, attn, out_proj_w.T,
                           out_proj_b.reshape(1, D), ln_gamma.reshape(1, D),
                           ln_beta.reshape(1, D), naff)
    return new2d.reshape(1, N_SLOTS, D), norm_aff
